# baseline (speedup 1.0000x reference)
import functools

import jax
import jax.numpy as jnp
from jax import lax
from jax.experimental import pallas as pl
from jax.experimental.pallas import tpu as pltpu
from jax.experimental.pallas import tpu_sc as plsc

N = 10000
NP = 10112
D = 128
E = 320000
NWORK = 32
EPT = 10240
GPT = EPT // 128
EPAD = NWORK * EPT
ROWS_PT = NP // 16
EPS = 1e-5

_mesh = plsc.VectorSubcoreMesh(core_axis_name="c", subcore_axis_name="s")


def _acc_chunks():
    off = 0
    for sz in (128, 128, 128, 128, 120):
        yield off, sz
        off += sz


@functools.partial(
    pl.kernel,
    out_type=jax.ShapeDtypeStruct((2, NP, D), jnp.float32),
    mesh=_mesh,
    scratch_types=[
        pltpu.VMEM((GPT // 5, 128), jnp.int32),
        pltpu.VMEM((GPT // 5, 128), jnp.int32),
        pltpu.VMEM((128, D), jnp.float32),
        pltpu.VMEM((128, D), jnp.float32),
        pltpu.VMEM_SHARED((NP, D), jnp.float32),
        pltpu.SemaphoreType.DMA,
        pltpu.SemaphoreType.DMA,
        pltpu.SemaphoreType.DMA,
        pltpu.SemaphoreType.DMA,
    ],
)
def _sc_prop(hp_hbm, src_hbm, dst_hbm, zeros_hbm, out_hbm,
             sidx_v, didx_v, rows_v, rowsB_v, acc_sh, semGA, semGB, semSA, semSB):
    c = lax.axis_index("c")
    s = lax.axis_index("s")
    wid = c * 16 + s
    pltpu.sync_copy(zeros_hbm, rows_v)
    base = s * ROWS_PT
    for off, sz in _acc_chunks():
        pltpu.sync_copy(rows_v.at[pl.ds(0, sz)], acc_sh.at[pl.ds(base + off, sz)])
    plsc.subcore_barrier()

    Q = GPT // 5
    bufs = (rows_v, rowsB_v)
    semG = (semGA, semGB)
    semS = (semSA, semSB)
    for quarter in range(5):
        pltpu.sync_copy(src_hbm.at[pl.ds(wid * GPT + quarter * Q, Q)], sidx_v)
        pltpu.sync_copy(dst_hbm.at[pl.ds(wid * GPT + quarter * Q, Q)], didx_v)

        def body(j, carry):
            for r in range(2):
                @pl.when((j >= 2) & (j % 2 == r))
                def _(_r=r):
                    pltpu.make_async_copy(bufs[_r], acc_sh.at[didx_v.at[j - 2]],
                                          semS[_r]).wait()

                @pl.when((j < Q) & (j % 2 == r))
                def _(_r=r):
                    pltpu.async_copy(hp_hbm.at[sidx_v.at[j]], bufs[_r], semG[_r])

            for r in range(2):
                @pl.when((j >= 1) & (j < Q + 1) & ((j - 1) % 2 == r))
                def _(_r=r):
                    pltpu.make_async_copy(hp_hbm.at[sidx_v.at[j - 1]],
                                          bufs[_r], semG[_r]).wait()
                    pltpu.async_copy(bufs[_r], acc_sh.at[didx_v.at[j - 1]],
                                     semS[_r], add=True)

            return carry

        lax.fori_loop(0, Q + 2, body, 0)

    plsc.subcore_barrier()
    for off, sz in _acc_chunks():
        pltpu.sync_copy(acc_sh.at[pl.ds(base + off, sz)], rows_v.at[pl.ds(0, sz)])
        pltpu.sync_copy(rows_v.at[pl.ds(0, sz)], out_hbm.at[c, pl.ds(base + off, sz)])


@functools.partial(
    pl.kernel,
    out_type=jax.ShapeDtypeStruct((2, NP, D), jnp.float32),
    mesh=_mesh,
    scratch_types=[
        pltpu.VMEM((GPT // 2, 128), jnp.int32),
        pltpu.VMEM((128, D), jnp.float32),
        pltpu.VMEM_SHARED((NP, D), jnp.float32),
        pltpu.SemaphoreType.DMA,
    ],
)
def _sc_deg(dst_hbm, ones_hbm, zeros_hbm, out_hbm, didx_v, ones_v, acc_sh, sem):
    c = lax.axis_index("c")
    s = lax.axis_index("s")
    wid = c * 16 + s
    pltpu.sync_copy(dst_hbm.at[pl.ds(wid * GPT, GPT // 2)], didx_v)
    pltpu.sync_copy(zeros_hbm, ones_v)
    base = s * ROWS_PT
    for off, sz in _acc_chunks():
        pltpu.sync_copy(ones_v.at[pl.ds(0, sz)], acc_sh.at[pl.ds(base + off, sz)])
    pltpu.sync_copy(ones_hbm, ones_v)
    plsc.subcore_barrier()

    def fire(j, carry):
        pltpu.async_copy(ones_v, acc_sh.at[didx_v.at[j]], sem, add=True)
        return carry

    def drain(j, carry):
        pltpu.make_async_copy(ones_v, acc_sh.at[didx_v.at[j]], sem).wait()
        return carry

    lax.fori_loop(0, GPT // 2, fire, 0)
    lax.fori_loop(0, GPT // 2, drain, 0)
    pltpu.sync_copy(dst_hbm.at[pl.ds(wid * GPT + GPT // 2, GPT // 2)], didx_v)
    lax.fori_loop(0, GPT // 2, fire, 0)
    lax.fori_loop(0, GPT // 2, drain, 0)
    plsc.subcore_barrier()
    for off, sz in _acc_chunks():
        pltpu.sync_copy(acc_sh.at[pl.ds(base + off, sz)], ones_v.at[pl.ds(0, sz)])
        pltpu.sync_copy(ones_v.at[pl.ds(0, sz)], out_hbm.at[c, pl.ds(base + off, sz)])


_R = 1000
_HIGH = lax.Precision.HIGHEST


def _tc_mm_body(x_ref, w_ref, o_ref):
    o_ref[...] = jnp.dot(x_ref[...], w_ref[...], precision=_HIGH,
                         preferred_element_type=jnp.float32)


def _tc_mm(x, w):
    m = x.shape[0]
    return pl.pallas_call(
        _tc_mm_body,
        grid=(m // _R,),
        in_specs=[
            pl.BlockSpec((_R, x.shape[1]), lambda i: (i, 0)),
            pl.BlockSpec(w.shape, lambda i: (0, 0)),
        ],
        out_specs=pl.BlockSpec((_R, w.shape[1]), lambda i: (i, 0)),
        out_shape=jax.ShapeDtypeStruct((m, w.shape[1]), jnp.float32),
    )(x, w)


def _tc_dinv_body(degp_ref, xw_ref, dinvb_ref, hp_ref):
    cnt = (degp_ref[0] + degp_ref[1])[:, :1] + 1.0
    dinv = lax.rsqrt(cnt)
    dinvb = jnp.broadcast_to(dinv, (_R, D))
    dinvb_ref[...] = dinvb
    hp_ref[...] = dinvb * xw_ref[...]


def _tc_dinv_scale(degp, xw):
    blk = pl.BlockSpec((_R, D), lambda i: (i, 0))
    return pl.pallas_call(
        _tc_dinv_body,
        grid=(N // _R,),
        in_specs=[pl.BlockSpec((2, _R, D), lambda i: (0, i, 0)), blk],
        out_specs=[blk, blk],
        out_shape=[jax.ShapeDtypeStruct((N, D), jnp.float32),
                   jax.ShapeDtypeStruct((N, D), jnp.float32)],
    )(degp, xw)


def _tc_layer_body(agg_ref, hp_ref, dinvb_ref, b_ref, g_ref, be_ref, w_ref,
                   x_out_ref, hp_out_ref):
    dinvb = dinvb_ref[...]
    agg = agg_ref[0] + agg_ref[1] + hp_ref[...]
    conv = dinvb * agg + b_ref[...]
    scale = g_ref[...] * lax.rsqrt(jnp.float32(1.0 + EPS))
    xl = jnp.maximum(conv * scale + be_ref[...], 0.0)
    x_out_ref[...] = xl
    hp_out_ref[...] = dinvb * jnp.dot(xl, w_ref[...], precision=_HIGH,
                                      preferred_element_type=jnp.float32)


def _tc_layer(aggp, hp, dinvb, b, g, be, w):
    return pl.pallas_call(
        _tc_layer_body,
        grid=(N // _R,),
        in_specs=[
            pl.BlockSpec((2, _R, D), lambda i: (0, i, 0)),
            pl.BlockSpec((_R, D), lambda i: (i, 0)),
            pl.BlockSpec((_R, D), lambda i: (i, 0)),
            pl.BlockSpec((1, D), lambda i: (0, 0)),
            pl.BlockSpec((1, D), lambda i: (0, 0)),
            pl.BlockSpec((1, D), lambda i: (0, 0)),
            pl.BlockSpec((D, D), lambda i: (0, 0)),
        ],
        out_specs=[
            pl.BlockSpec((_R, D), lambda i: (i, 0)),
            pl.BlockSpec((_R, D), lambda i: (i, 0)),
        ],
        out_shape=[
            jax.ShapeDtypeStruct((N, D), jnp.float32),
            jax.ShapeDtypeStruct((N, D), jnp.float32),
        ],
    )(aggp, hp, dinvb, b, g, be, w)


def _tc_final_body(agg_ref, hp_ref, dinvb_ref, b_ref, g_ref, be_ref,
                   x1_ref, lw1a_ref, lw1b_ref, lb1_ref, lw2_ref, lb2_ref,
                   o_ref):
    dinvb = dinvb_ref[...]
    agg = agg_ref[0] + agg_ref[1] + hp_ref[...]
    conv = dinvb * agg + b_ref[...]
    scale = g_ref[...] * lax.rsqrt(jnp.float32(1.0 + EPS))
    x2 = jnp.maximum(conv * scale + be_ref[...], 0.0)
    h = jnp.dot(x1_ref[...], lw1a_ref[...], precision=_HIGH,
                preferred_element_type=jnp.float32)
    h += jnp.dot(x2, lw1b_ref[...], precision=_HIGH,
                 preferred_element_type=jnp.float32)
    h = jnp.maximum(h + lb1_ref[...], 0.0)
    o_ref[...] = jnp.dot(h, lw2_ref[...], precision=_HIGH,
                         preferred_element_type=jnp.float32) + lb2_ref[...]


def _tc_final(aggp, hp, dinvb, b, g, be, x1, lw1a, lw1b, lb1, lw2, lb2):
    vec = pl.BlockSpec((1, D), lambda i: (0, 0))
    mat = pl.BlockSpec((D, D), lambda i: (0, 0))
    blk = pl.BlockSpec((_R, D), lambda i: (i, 0))
    return pl.pallas_call(
        _tc_final_body,
        grid=(N // _R,),
        in_specs=[pl.BlockSpec((2, _R, D), lambda i: (0, i, 0)),
                  blk, blk, vec, vec, vec, blk, mat, mat, vec, mat, vec],
        out_specs=blk,
        out_shape=jax.ShapeDtypeStruct((N, D), jnp.float32),
    )(aggp, hp, dinvb, b, g, be, x1, lw1a, lw1b, lb1, lw2, lb2)


def kernel(x, edge_index, W1, b1, g1, be1, W2, b2, g2, be2, LW1, Lb1, LW2, Lb2):
    src = edge_index[0]
    dst = edge_index[1]
    pad = EPAD - E
    ar = jnp.arange(pad, dtype=jnp.int32)
    src_p = jnp.concatenate([src, (ar * 79) % N]).reshape(EPAD // 128, 128)
    dst_p = jnp.concatenate([dst, N + ar % (NP - N)]).reshape(EPAD // 128, 128)
    zerosD = jnp.zeros((128, D), jnp.float32)
    onesND = jnp.ones((N, D), jnp.float32)

    degp = _sc_deg(dst_p, onesND[:128], zerosD)
    xw1 = _tc_mm(x, W1)
    dinvb, h1p = _tc_dinv_scale(degp, xw1)
    agg1 = _sc_prop(h1p, src_p, dst_p, zerosD)
    x1, h2p = _tc_layer(agg1, h1p, dinvb,
                        b1.reshape(1, D), g1.reshape(1, D), be1.reshape(1, D), W2)
    agg2 = _sc_prop(h2p, src_p, dst_p, zerosD)
    out = _tc_final(agg2, h2p, dinvb,
                    b2.reshape(1, D), g2.reshape(1, D), be2.reshape(1, D),
                    x1, LW1[:D], LW1[D:], Lb1.reshape(1, D), LW2, Lb2.reshape(1, D))
    return out


# speedup vs baseline: 1.0426x; 1.0426x over previous
import functools

import jax
import jax.numpy as jnp
from jax import lax
from jax.experimental import pallas as pl
from jax.experimental.pallas import tpu as pltpu
from jax.experimental.pallas import tpu_sc as plsc

N = 10000
NP = 10112
D = 128
E = 320000
NWORK = 32
EPT = 10240
GPT = EPT // 128
EPAD = NWORK * EPT
ROWS_PT = NP // 16
EPS = 1e-5

_mesh = plsc.VectorSubcoreMesh(core_axis_name="c", subcore_axis_name="s")


def _acc_chunks():
    off = 0
    for sz in (128, 128, 128, 128, 120):
        yield off, sz
        off += sz


@functools.partial(
    pl.kernel,
    out_type=jax.ShapeDtypeStruct((2, NP, D), jnp.float32),
    mesh=_mesh,
    scratch_types=[
        pltpu.VMEM((GPT // 2, 128), jnp.int32),
        pltpu.VMEM((GPT, 128), jnp.int32),
        pltpu.VMEM((128, D), jnp.float32),
        pltpu.VMEM((128, D), jnp.float32),
        pltpu.VMEM_SHARED((NP, D), jnp.float32),
        pltpu.SemaphoreType.DMA,
        pltpu.SemaphoreType.DMA,
    ],
)
def _sc_prop(hp_hbm, src_hbm, dst_hbm, zeros_hbm, out_hbm,
             sidx_v, didx_v, rows_v, rowsB_v, acc_sh, sem, semB):
    c = lax.axis_index("c")
    s = lax.axis_index("s")
    wid = c * 16 + s
    pltpu.sync_copy(dst_hbm.at[pl.ds(wid * GPT, GPT)], didx_v)
    pltpu.sync_copy(zeros_hbm, rows_v)
    base = s * ROWS_PT
    for off, sz in _acc_chunks():
        pltpu.sync_copy(rows_v.at[pl.ds(0, sz)], acc_sh.at[pl.ds(base + off, sz)])
    plsc.subcore_barrier()

    H = GPT // 2
    for half in range(2):
        pltpu.sync_copy(src_hbm.at[pl.ds(wid * GPT + half * H, H)], sidx_v)

        def body(j, carry, _hb=half * H):
            @pl.when((j < H) & (j % 2 == 0))
            def _():
                pltpu.async_copy(hp_hbm.at[sidx_v.at[j]], rows_v, sem)

            @pl.when((j < H) & (j % 2 == 1))
            def _():
                pltpu.async_copy(hp_hbm.at[sidx_v.at[j]], rowsB_v, semB)

            @pl.when((j >= 1) & (j % 2 == 1))
            def _():
                pltpu.make_async_copy(hp_hbm.at[sidx_v.at[j - 1]], rows_v, sem).wait()
                pltpu.sync_copy(rows_v, acc_sh.at[didx_v.at[_hb + j - 1]], add=True)

            @pl.when((j >= 1) & (j % 2 == 0))
            def _():
                pltpu.make_async_copy(hp_hbm.at[sidx_v.at[j - 1]], rowsB_v, semB).wait()
                pltpu.sync_copy(rowsB_v, acc_sh.at[didx_v.at[_hb + j - 1]], add=True)

            return carry

        lax.fori_loop(0, H + 1, body, 0)
    plsc.subcore_barrier()
    for off, sz in _acc_chunks():
        pltpu.sync_copy(acc_sh.at[pl.ds(base + off, sz)], rows_v.at[pl.ds(0, sz)])
        pltpu.sync_copy(rows_v.at[pl.ds(0, sz)], out_hbm.at[c, pl.ds(base + off, sz)])


@functools.partial(
    pl.kernel,
    out_type=jax.ShapeDtypeStruct((2, NP, D), jnp.float32),
    mesh=_mesh,
    scratch_types=[
        pltpu.VMEM((GPT // 2, 128), jnp.int32),
        pltpu.VMEM((128, D), jnp.float32),
        pltpu.VMEM_SHARED((NP, D), jnp.float32),
        pltpu.SemaphoreType.DMA,
    ],
)
def _sc_deg(dst_hbm, ones_hbm, zeros_hbm, out_hbm, didx_v, ones_v, acc_sh, sem):
    c = lax.axis_index("c")
    s = lax.axis_index("s")
    wid = c * 16 + s
    pltpu.sync_copy(dst_hbm.at[pl.ds(wid * GPT, GPT // 2)], didx_v)
    pltpu.sync_copy(zeros_hbm, ones_v)
    base = s * ROWS_PT
    for off, sz in _acc_chunks():
        pltpu.sync_copy(ones_v.at[pl.ds(0, sz)], acc_sh.at[pl.ds(base + off, sz)])
    pltpu.sync_copy(ones_hbm, ones_v)
    plsc.subcore_barrier()

    def fire(j, carry):
        pltpu.async_copy(ones_v, acc_sh.at[didx_v.at[j]], sem, add=True)
        return carry

    def drain(j, carry):
        pltpu.make_async_copy(ones_v, acc_sh.at[didx_v.at[j]], sem).wait()
        return carry

    lax.fori_loop(0, GPT // 2, fire, 0)
    lax.fori_loop(0, GPT // 2, drain, 0)
    pltpu.sync_copy(dst_hbm.at[pl.ds(wid * GPT + GPT // 2, GPT // 2)], didx_v)
    lax.fori_loop(0, GPT // 2, fire, 0)
    lax.fori_loop(0, GPT // 2, drain, 0)
    plsc.subcore_barrier()
    for off, sz in _acc_chunks():
        pltpu.sync_copy(acc_sh.at[pl.ds(base + off, sz)], ones_v.at[pl.ds(0, sz)])
        pltpu.sync_copy(ones_v.at[pl.ds(0, sz)], out_hbm.at[c, pl.ds(base + off, sz)])


_R = 1000
_HIGH = lax.Precision.HIGHEST


def _tc_mm_body(x_ref, w_ref, o_ref):
    o_ref[...] = jnp.dot(x_ref[...], w_ref[...], precision=_HIGH,
                         preferred_element_type=jnp.float32)


def _tc_mm(x, w):
    m = x.shape[0]
    return pl.pallas_call(
        _tc_mm_body,
        grid=(m // _R,),
        in_specs=[
            pl.BlockSpec((_R, x.shape[1]), lambda i: (i, 0)),
            pl.BlockSpec(w.shape, lambda i: (0, 0)),
        ],
        out_specs=pl.BlockSpec((_R, w.shape[1]), lambda i: (i, 0)),
        out_shape=jax.ShapeDtypeStruct((m, w.shape[1]), jnp.float32),
    )(x, w)


def _tc_dinv_body(degp_ref, xw_ref, dinvb_ref, hp_ref):
    cnt = (degp_ref[0] + degp_ref[1])[:, :1] + 1.0
    dinv = lax.rsqrt(cnt)
    dinvb = jnp.broadcast_to(dinv, (_R, D))
    dinvb_ref[...] = dinvb
    hp_ref[...] = dinvb * xw_ref[...]


def _tc_dinv_scale(degp, xw):
    blk = pl.BlockSpec((_R, D), lambda i: (i, 0))
    return pl.pallas_call(
        _tc_dinv_body,
        grid=(N // _R,),
        in_specs=[pl.BlockSpec((2, _R, D), lambda i: (0, i, 0)), blk],
        out_specs=[blk, blk],
        out_shape=[jax.ShapeDtypeStruct((N, D), jnp.float32),
                   jax.ShapeDtypeStruct((N, D), jnp.float32)],
    )(degp, xw)


def _tc_layer_body(agg_ref, hp_ref, dinvb_ref, b_ref, g_ref, be_ref, w_ref,
                   x_out_ref, hp_out_ref):
    dinvb = dinvb_ref[...]
    agg = agg_ref[0] + agg_ref[1] + hp_ref[...]
    conv = dinvb * agg + b_ref[...]
    scale = g_ref[...] * lax.rsqrt(jnp.float32(1.0 + EPS))
    xl = jnp.maximum(conv * scale + be_ref[...], 0.0)
    x_out_ref[...] = xl
    hp_out_ref[...] = dinvb * jnp.dot(xl, w_ref[...], precision=_HIGH,
                                      preferred_element_type=jnp.float32)


def _tc_layer(aggp, hp, dinvb, b, g, be, w):
    return pl.pallas_call(
        _tc_layer_body,
        grid=(N // _R,),
        in_specs=[
            pl.BlockSpec((2, _R, D), lambda i: (0, i, 0)),
            pl.BlockSpec((_R, D), lambda i: (i, 0)),
            pl.BlockSpec((_R, D), lambda i: (i, 0)),
            pl.BlockSpec((1, D), lambda i: (0, 0)),
            pl.BlockSpec((1, D), lambda i: (0, 0)),
            pl.BlockSpec((1, D), lambda i: (0, 0)),
            pl.BlockSpec((D, D), lambda i: (0, 0)),
        ],
        out_specs=[
            pl.BlockSpec((_R, D), lambda i: (i, 0)),
            pl.BlockSpec((_R, D), lambda i: (i, 0)),
        ],
        out_shape=[
            jax.ShapeDtypeStruct((N, D), jnp.float32),
            jax.ShapeDtypeStruct((N, D), jnp.float32),
        ],
    )(aggp, hp, dinvb, b, g, be, w)


def _tc_final_body(agg_ref, hp_ref, dinvb_ref, b_ref, g_ref, be_ref,
                   x1_ref, lw1a_ref, lw1b_ref, lb1_ref, lw2_ref, lb2_ref,
                   o_ref):
    dinvb = dinvb_ref[...]
    agg = agg_ref[0] + agg_ref[1] + hp_ref[...]
    conv = dinvb * agg + b_ref[...]
    scale = g_ref[...] * lax.rsqrt(jnp.float32(1.0 + EPS))
    x2 = jnp.maximum(conv * scale + be_ref[...], 0.0)
    h = jnp.dot(x1_ref[...], lw1a_ref[...], precision=_HIGH,
                preferred_element_type=jnp.float32)
    h += jnp.dot(x2, lw1b_ref[...], precision=_HIGH,
                 preferred_element_type=jnp.float32)
    h = jnp.maximum(h + lb1_ref[...], 0.0)
    o_ref[...] = jnp.dot(h, lw2_ref[...], precision=_HIGH,
                         preferred_element_type=jnp.float32) + lb2_ref[...]


def _tc_final(aggp, hp, dinvb, b, g, be, x1, lw1a, lw1b, lb1, lw2, lb2):
    vec = pl.BlockSpec((1, D), lambda i: (0, 0))
    mat = pl.BlockSpec((D, D), lambda i: (0, 0))
    blk = pl.BlockSpec((_R, D), lambda i: (i, 0))
    return pl.pallas_call(
        _tc_final_body,
        grid=(N // _R,),
        in_specs=[pl.BlockSpec((2, _R, D), lambda i: (0, i, 0)),
                  blk, blk, vec, vec, vec, blk, mat, mat, vec, mat, vec],
        out_specs=blk,
        out_shape=jax.ShapeDtypeStruct((N, D), jnp.float32),
    )(aggp, hp, dinvb, b, g, be, x1, lw1a, lw1b, lb1, lw2, lb2)


def kernel(x, edge_index, W1, b1, g1, be1, W2, b2, g2, be2, LW1, Lb1, LW2, Lb2):
    src = edge_index[0]
    dst = edge_index[1]
    pad = EPAD - E
    ar = jnp.arange(pad, dtype=jnp.int32)
    src_p = jnp.concatenate([src, (ar * 79) % N]).reshape(EPAD // 128, 128)
    dst_p = jnp.concatenate([dst, N + ar % (NP - N)]).reshape(EPAD // 128, 128)
    zerosD = jnp.zeros((128, D), jnp.float32)
    onesND = jnp.ones((N, D), jnp.float32)

    degp = _sc_deg(dst_p, onesND[:128], zerosD)
    xw1 = _tc_mm(x, W1)
    dinvb, h1p = _tc_dinv_scale(degp, xw1)
    agg1 = _sc_prop(h1p, src_p, dst_p, zerosD)
    x1, h2p = _tc_layer(agg1, h1p, dinvb,
                        b1.reshape(1, D), g1.reshape(1, D), be1.reshape(1, D), W2)
    agg2 = _sc_prop(h2p, src_p, dst_p, zerosD)
    out = _tc_final(agg2, h2p, dinvb,
                    b2.reshape(1, D), g2.reshape(1, D), be2.reshape(1, D),
                    x1, LW1[:D], LW1[D:], Lb1.reshape(1, D), LW2, Lb2.reshape(1, D))
    return out
